# baseline (speedup 1.0000x reference)
import functools

import jax
import jax.numpy as jnp
from jax import lax
from jax.experimental import pallas as pl
from jax.experimental.pallas import tpu as pltpu
from jax.experimental.pallas import tpu_sc as plsc


@functools.lru_cache(maxsize=None)
def _build_pack_table(C, V):
    BN = 4096
    SB = 128

    def body(in_ref, out_ref):
        a = in_ref[...]
        jj = lax.broadcasted_iota(jnp.int32, (SB // 2, SB), 1)
        pp = lax.broadcasted_iota(jnp.int32, (SB // 2, SB), 0)
        l_e = (jj == 2 * pp).astype(jnp.float32)
        l_o = (jj == 2 * pp + 1).astype(jnp.float32)
        dd = lax.broadcasted_iota(jnp.int32, (C, 2 * C), 0)
        cc = lax.broadcasted_iota(jnp.int32, (C, 2 * C), 1)
        r_lo = (cc == dd).astype(jnp.float32)
        r_hi = (cc == dd + C).astype(jnp.float32)
        dn = (((1,), (1,)), ((), ()))
        dn2 = (((1,), (0,)), ((), ()))
        hi = lax.Precision.HIGHEST
        for k in range(BN // SB):
            sub = a[:, k * SB:(k + 1) * SB]
            e = lax.dot_general(l_e, sub, dn, precision=hi)
            o = lax.dot_general(l_o, sub, dn, precision=hi)
            out_ref[pl.ds(k * SB // 2, SB // 2), :] = (
                lax.dot_general(e, r_lo, dn2, precision=hi)
                + lax.dot_general(o, r_hi, dn2, precision=hi))

    return pl.pallas_call(
        body,
        grid=(pl.cdiv(V, BN),),
        in_specs=[pl.BlockSpec((C, BN), lambda i: (0, i))],
        out_specs=pl.BlockSpec((BN // 2, 2 * C), lambda i: (i, 0)),
        out_shape=jax.ShapeDtypeStruct((V // 2, 2 * C), jnp.float32),
    )


@functools.lru_cache(maxsize=None)
def _build_transpose(M, K):
    L = 128
    BM = 512
    nk = K // L
    assert M % BM == 0 and K % L == 0

    def body(in_ref, out_ref):
        a = in_ref[...].reshape(BM, nk, L)
        for ks in range(nk):
            out_ref[pl.ds(ks * L, L), :] = jnp.transpose(a[:, ks, :])

    return pl.pallas_call(
        body,
        grid=(M // BM,),
        in_specs=[pl.BlockSpec((BM * nk, L), lambda i: (i, 0))],
        out_specs=pl.BlockSpec((K, BM), lambda i: (0, i)),
        out_shape=jax.ShapeDtypeStruct((K, M), jnp.float32),
    )


@functools.lru_cache(maxsize=None)
def _build_gather(B, V, D):
    info = plsc.get_sparse_core_info()
    NC, NS = info.num_cores, info.num_subcores
    NW = NC * NS
    assert B % NW == 0
    b_per_w = B // NW
    CH = 256
    NBUF = 4
    assert b_per_w % CH == 0
    n_ch = b_per_w // CH
    assert n_ch % NBUF == 0

    mesh = plsc.VectorSubcoreMesh(core_axis_name="c", subcore_axis_name="s")

    @functools.partial(
        pl.kernel,
        mesh=mesh,
        compiler_params=pltpu.CompilerParams(use_tc_tiling_on_sc=False),
        out_type=jax.ShapeDtypeStruct((B, D), jnp.float32),
        scratch_types=(
            [pltpu.VMEM((b_per_w,), jnp.int32),
             pltpu.VMEM((NBUF, CH, D), jnp.float32)]
            + [pltpu.SemaphoreType.DMA] * (2 * NBUF)
        ),
    )
    def gather_kernel(idx_hbm, table_hbm, out_hbm, idx_v, rows_v, *sems):
        gsems, psems = sems[:NBUF], sems[NBUF:]
        wid = lax.axis_index("s") * NC + lax.axis_index("c")
        base = wid * b_per_w
        pltpu.sync_copy(idx_hbm.at[pl.ds(base, b_per_w)], idx_v)

        def start_gather(j, b):
            pltpu.async_copy(
                table_hbm.at[idx_v.at[pl.ds(j * CH, CH)]], rows_v.at[b], gsems[b])

        def wait_gather(b):
            pltpu.make_async_copy(
                table_hbm.at[pl.ds(0, CH)], rows_v.at[b], gsems[b]).wait()

        def start_put(j, b):
            pltpu.async_copy(
                rows_v.at[b], out_hbm.at[pl.ds(base + j * CH, CH)], psems[b])

        def wait_put(b):
            pltpu.make_async_copy(
                rows_v.at[b], out_hbm.at[pl.ds(0, CH)], psems[b]).wait()

        for j in range(NBUF - 1):
            start_gather(j, j)

        def group(g, carry):
            for b in range(NBUF):
                j = g * NBUF + b
                wait_gather(b)
                start_put(j, b)
                gj = j + NBUF - 1
                gb = (b - 1) % NBUF

                @pl.when(gj < n_ch)
                def _():
                    @pl.when(j > 0)
                    def _():
                        wait_put(gb)
                    start_gather(gj, gb)
            return carry

        lax.fori_loop(0, n_ch // NBUF, group, 0)

        for b in range(NBUF):
            wait_put(b)

    return gather_kernel


def kernel(x, table):
    V, D = table.shape
    N, S = x.shape
    B = N * S
    xf = x.reshape(-1).astype(jnp.int32)
    t_pair = _build_pack_table(D, V)(jnp.transpose(table))
    t_lin = jnp.reshape(t_pair, (V, D))
    res = _build_gather(B, V, D)(xf, t_lin)
    K = S * D
    in2 = jnp.reshape(res, (B * D // 128, 128))
    out2 = _build_transpose(N, K)(in2)
    out3 = jnp.reshape(out2, (S, D, N))
    return jnp.transpose(out3, (2, 0, 1))


# speedup vs baseline: 3.8682x; 3.8682x over previous
import functools

import jax
import jax.numpy as jnp
from jax import lax
from jax.experimental import pallas as pl
from jax.experimental.pallas import tpu as pltpu
from jax.experimental.pallas import tpu_sc as plsc


@functools.lru_cache(maxsize=None)
def _build_pack_table(C, V):
    BN = 4096
    SB = 128

    def body(in_ref, out_ref):
        a = in_ref[...]
        t = jnp.transpose(a)
        t3 = t.reshape(BN // 2, 2, C)
        out_ref[...] = jnp.concatenate([t3[:, 0, :], t3[:, 1, :]], axis=1)

    return pl.pallas_call(
        body,
        grid=(pl.cdiv(V, BN),),
        in_specs=[pl.BlockSpec((C, BN), lambda i: (0, i))],
        out_specs=pl.BlockSpec((BN // 2, 2 * C), lambda i: (i, 0)),
        out_shape=jax.ShapeDtypeStruct((V // 2, 2 * C), jnp.float32),
    )


@functools.lru_cache(maxsize=None)
def _build_transpose(M, K):
    L = 128
    BM = 512
    nk = K // L
    assert M % BM == 0 and K % L == 0

    def body(in_ref, out_ref):
        a = in_ref[...].reshape(BM, nk, L)
        for ks in range(nk):
            out_ref[pl.ds(ks * L, L), :] = jnp.transpose(a[:, ks, :])

    return pl.pallas_call(
        body,
        grid=(M // BM,),
        in_specs=[pl.BlockSpec((BM * nk, L), lambda i: (i, 0))],
        out_specs=pl.BlockSpec((K, BM), lambda i: (0, i)),
        out_shape=jax.ShapeDtypeStruct((K, M), jnp.float32),
    )


@functools.lru_cache(maxsize=None)
def _build_gather(B, V, D):
    info = plsc.get_sparse_core_info()
    NC, NS = info.num_cores, info.num_subcores
    NW = NC * NS
    assert B % NW == 0
    b_per_w = B // NW
    CH = 256
    NBUF = 4
    assert b_per_w % CH == 0
    n_ch = b_per_w // CH
    assert n_ch % NBUF == 0

    mesh = plsc.VectorSubcoreMesh(core_axis_name="c", subcore_axis_name="s")

    @functools.partial(
        pl.kernel,
        mesh=mesh,
        compiler_params=pltpu.CompilerParams(use_tc_tiling_on_sc=False),
        out_type=jax.ShapeDtypeStruct((B, D), jnp.float32),
        scratch_types=(
            [pltpu.VMEM((b_per_w,), jnp.int32),
             pltpu.VMEM((NBUF, CH, D), jnp.float32)]
            + [pltpu.SemaphoreType.DMA] * (2 * NBUF)
        ),
    )
    def gather_kernel(idx_hbm, table_hbm, out_hbm, idx_v, rows_v, *sems):
        gsems, psems = sems[:NBUF], sems[NBUF:]
        wid = lax.axis_index("s") * NC + lax.axis_index("c")
        base = wid * b_per_w
        pltpu.sync_copy(idx_hbm.at[pl.ds(base, b_per_w)], idx_v)

        def start_gather(j, b):
            pltpu.async_copy(
                table_hbm.at[idx_v.at[pl.ds(j * CH, CH)]], rows_v.at[b], gsems[b])

        def wait_gather(b):
            pltpu.make_async_copy(
                table_hbm.at[pl.ds(0, CH)], rows_v.at[b], gsems[b]).wait()

        def start_put(j, b):
            pltpu.async_copy(
                rows_v.at[b], out_hbm.at[pl.ds(base + j * CH, CH)], psems[b])

        def wait_put(b):
            pltpu.make_async_copy(
                rows_v.at[b], out_hbm.at[pl.ds(0, CH)], psems[b]).wait()

        for j in range(NBUF - 1):
            start_gather(j, j)

        def group(g, carry):
            for b in range(NBUF):
                j = g * NBUF + b
                wait_gather(b)
                start_put(j, b)
                gj = j + NBUF - 1
                gb = (b - 1) % NBUF

                @pl.when(gj < n_ch)
                def _():
                    @pl.when(j > 0)
                    def _():
                        wait_put(gb)
                    start_gather(gj, gb)
            return carry

        lax.fori_loop(0, n_ch // NBUF, group, 0)

        for b in range(NBUF):
            wait_put(b)

    return gather_kernel


def kernel(x, table):
    V, D = table.shape
    N, S = x.shape
    B = N * S
    xf = x.reshape(-1).astype(jnp.int32)
    t_pair = _build_pack_table(D, V)(jnp.transpose(table))
    t_lin = jnp.reshape(t_pair, (V, D))
    res = _build_gather(B, V, D)(xf, t_lin)
    K = S * D
    in2 = jnp.reshape(res, (B * D // 128, 128))
    out2 = _build_transpose(N, K)(in2)
    out3 = jnp.reshape(out2, (S, D, N))
    return jnp.transpose(out3, (2, 0, 1))


# speedup vs baseline: 3.8688x; 1.0002x over previous
import functools

import jax
import jax.numpy as jnp
from jax import lax
from jax.experimental import pallas as pl
from jax.experimental.pallas import tpu as pltpu
from jax.experimental.pallas import tpu_sc as plsc


@functools.lru_cache(maxsize=None)
def _build_pack_table(C, V):
    BN = 4096
    SB = 128

    def body(in_ref, out_ref):
        a = in_ref[...]
        t = jnp.transpose(a)
        t3 = t.reshape(BN // 2, 2, C)
        out_ref[:, 0:C] = t3[:, 0, :]
        out_ref[:, C:2 * C] = t3[:, 1, :]

    return pl.pallas_call(
        body,
        grid=(pl.cdiv(V, BN),),
        in_specs=[pl.BlockSpec((C, BN), lambda i: (0, i))],
        out_specs=pl.BlockSpec((BN // 2, 2 * C), lambda i: (i, 0)),
        out_shape=jax.ShapeDtypeStruct((V // 2, 2 * C), jnp.float32),
    )


@functools.lru_cache(maxsize=None)
def _build_transpose(M, K):
    L = 128
    BM = 512
    nk = K // L
    assert M % BM == 0 and K % L == 0

    def body(in_ref, out_ref):
        a = in_ref[...].reshape(BM, nk, L)
        for ks in range(nk):
            out_ref[pl.ds(ks * L, L), :] = jnp.transpose(a[:, ks, :])

    return pl.pallas_call(
        body,
        grid=(M // BM,),
        in_specs=[pl.BlockSpec((BM * nk, L), lambda i: (i, 0))],
        out_specs=pl.BlockSpec((K, BM), lambda i: (0, i)),
        out_shape=jax.ShapeDtypeStruct((K, M), jnp.float32),
    )


@functools.lru_cache(maxsize=None)
def _build_gather(B, V, D):
    info = plsc.get_sparse_core_info()
    NC, NS = info.num_cores, info.num_subcores
    NW = NC * NS
    assert B % NW == 0
    b_per_w = B // NW
    CH = 256
    NBUF = 4
    assert b_per_w % CH == 0
    n_ch = b_per_w // CH
    assert n_ch % NBUF == 0

    mesh = plsc.VectorSubcoreMesh(core_axis_name="c", subcore_axis_name="s")

    @functools.partial(
        pl.kernel,
        mesh=mesh,
        compiler_params=pltpu.CompilerParams(use_tc_tiling_on_sc=False),
        out_type=jax.ShapeDtypeStruct((B, D), jnp.float32),
        scratch_types=(
            [pltpu.VMEM((b_per_w,), jnp.int32),
             pltpu.VMEM((NBUF, CH, D), jnp.float32)]
            + [pltpu.SemaphoreType.DMA] * (2 * NBUF)
        ),
    )
    def gather_kernel(idx_hbm, table_hbm, out_hbm, idx_v, rows_v, *sems):
        gsems, psems = sems[:NBUF], sems[NBUF:]
        wid = lax.axis_index("s") * NC + lax.axis_index("c")
        base = wid * b_per_w
        pltpu.sync_copy(idx_hbm.at[pl.ds(base, b_per_w)], idx_v)

        def start_gather(j, b):
            pltpu.async_copy(
                table_hbm.at[idx_v.at[pl.ds(j * CH, CH)]], rows_v.at[b], gsems[b])

        def wait_gather(b):
            pltpu.make_async_copy(
                table_hbm.at[pl.ds(0, CH)], rows_v.at[b], gsems[b]).wait()

        def start_put(j, b):
            pltpu.async_copy(
                rows_v.at[b], out_hbm.at[pl.ds(base + j * CH, CH)], psems[b])

        def wait_put(b):
            pltpu.make_async_copy(
                rows_v.at[b], out_hbm.at[pl.ds(0, CH)], psems[b]).wait()

        for j in range(NBUF - 1):
            start_gather(j, j)

        def group(g, carry):
            for b in range(NBUF):
                j = g * NBUF + b
                wait_gather(b)
                start_put(j, b)
                gj = j + NBUF - 1
                gb = (b - 1) % NBUF

                @pl.when(gj < n_ch)
                def _():
                    @pl.when(j > 0)
                    def _():
                        wait_put(gb)
                    start_gather(gj, gb)
            return carry

        lax.fori_loop(0, n_ch // NBUF, group, 0)

        for b in range(NBUF):
            wait_put(b)

    return gather_kernel


def kernel(x, table):
    V, D = table.shape
    N, S = x.shape
    B = N * S
    xf = x.reshape(-1).astype(jnp.int32)
    t_pair = _build_pack_table(D, V)(jnp.transpose(table))
    t_lin = jnp.reshape(t_pair, (V, D))
    res = _build_gather(B, V, D)(xf, t_lin)
    K = S * D
    in2 = jnp.reshape(res, (B * D // 128, 128))
    out2 = _build_transpose(N, K)(in2)
    out3 = jnp.reshape(out2, (S, D, N))
    return jnp.transpose(out3, (2, 0, 1))


# speedup vs baseline: 4.0406x; 1.0444x over previous
import functools

import jax
import jax.numpy as jnp
from jax import lax
from jax.experimental import pallas as pl
from jax.experimental.pallas import tpu as pltpu
from jax.experimental.pallas import tpu_sc as plsc


@functools.lru_cache(maxsize=None)
def _build_pack_table(C, V):
    BN = 8192

    def body(in_ref, out_ref):
        a = in_ref[...]
        t = jnp.transpose(a)
        t3 = t.reshape(BN // 2, 2, C)
        out_ref[:, 0:C] = t3[:, 0, :]
        out_ref[:, C:2 * C] = t3[:, 1, :]

    return pl.pallas_call(
        body,
        grid=(pl.cdiv(V, BN),),
        in_specs=[pl.BlockSpec((C, BN), lambda i: (0, i))],
        out_specs=pl.BlockSpec((BN // 2, 2 * C), lambda i: (i, 0)),
        out_shape=jax.ShapeDtypeStruct((V // 2, 2 * C), jnp.float32),
    )


@functools.lru_cache(maxsize=None)
def _build_transpose(M, K):
    L = 128
    BM = 512
    nk = K // L
    assert M % BM == 0 and K % L == 0

    def body(in_ref, out_ref):
        a = in_ref[...].reshape(BM, nk, L)
        for ks in range(nk):
            out_ref[pl.ds(ks * L, L), :] = jnp.transpose(a[:, ks, :])

    return pl.pallas_call(
        body,
        grid=(M // BM,),
        in_specs=[pl.BlockSpec((BM * nk, L), lambda i: (i, 0))],
        out_specs=pl.BlockSpec((K, BM), lambda i: (0, i)),
        out_shape=jax.ShapeDtypeStruct((K, M), jnp.float32),
    )


@functools.lru_cache(maxsize=None)
def _build_gather(B, V, D):
    info = plsc.get_sparse_core_info()
    NC, NS = info.num_cores, info.num_subcores
    NW = NC * NS
    assert B % NW == 0
    b_per_w = B // NW
    CH = 256
    NBUF = 4
    assert b_per_w % CH == 0
    n_ch = b_per_w // CH
    assert n_ch % NBUF == 0

    mesh = plsc.VectorSubcoreMesh(core_axis_name="c", subcore_axis_name="s")

    @functools.partial(
        pl.kernel,
        mesh=mesh,
        compiler_params=pltpu.CompilerParams(use_tc_tiling_on_sc=False),
        out_type=jax.ShapeDtypeStruct((B, D), jnp.float32),
        scratch_types=(
            [pltpu.VMEM((b_per_w,), jnp.int32),
             pltpu.VMEM((NBUF, CH, D), jnp.float32)]
            + [pltpu.SemaphoreType.DMA] * (2 * NBUF)
        ),
    )
    def gather_kernel(idx_hbm, table_hbm, out_hbm, idx_v, rows_v, *sems):
        gsems, psems = sems[:NBUF], sems[NBUF:]
        wid = lax.axis_index("s") * NC + lax.axis_index("c")
        base = wid * b_per_w
        pltpu.sync_copy(idx_hbm.at[pl.ds(base, b_per_w)], idx_v)

        def start_gather(j, b):
            pltpu.async_copy(
                table_hbm.at[idx_v.at[pl.ds(j * CH, CH)]], rows_v.at[b], gsems[b])

        def wait_gather(b):
            pltpu.make_async_copy(
                table_hbm.at[pl.ds(0, CH)], rows_v.at[b], gsems[b]).wait()

        def start_put(j, b):
            pltpu.async_copy(
                rows_v.at[b], out_hbm.at[pl.ds(base + j * CH, CH)], psems[b])

        def wait_put(b):
            pltpu.make_async_copy(
                rows_v.at[b], out_hbm.at[pl.ds(0, CH)], psems[b]).wait()

        for j in range(NBUF - 1):
            start_gather(j, j)

        def group(g, carry):
            for b in range(NBUF):
                j = g * NBUF + b
                wait_gather(b)
                start_put(j, b)
                gj = j + NBUF - 1
                gb = (b - 1) % NBUF

                @pl.when(gj < n_ch)
                def _():
                    @pl.when(j > 0)
                    def _():
                        wait_put(gb)
                    start_gather(gj, gb)
            return carry

        lax.fori_loop(0, n_ch // NBUF, group, 0)

        for b in range(NBUF):
            wait_put(b)

    return gather_kernel


def kernel(x, table):
    V, D = table.shape
    N, S = x.shape
    B = N * S
    xf = x.reshape(-1).astype(jnp.int32)
    t_pair = _build_pack_table(D, V)(jnp.transpose(table))
    t_lin = jnp.reshape(t_pair, (V, D))
    res = _build_gather(B, V, D)(xf, t_lin)
    K = S * D
    in2 = jnp.reshape(res, (B * D // 128, 128))
    out2 = _build_transpose(N, K)(in2)
    out3 = jnp.reshape(out2, (S, D, N))
    return jnp.transpose(out3, (2, 0, 1))


# speedup vs baseline: 4.0740x; 1.0083x over previous
import functools

import jax
import jax.numpy as jnp
from jax import lax
from jax.experimental import pallas as pl
from jax.experimental.pallas import tpu as pltpu
from jax.experimental.pallas import tpu_sc as plsc


@functools.lru_cache(maxsize=None)
def _build_pack_table(C, V):
    BN = 16384

    def body(in_ref, out_ref):
        a = in_ref[...]
        t = jnp.transpose(a)
        t3 = t.reshape(BN // 2, 2, C)
        out_ref[:, 0:C] = t3[:, 0, :]
        out_ref[:, C:2 * C] = t3[:, 1, :]

    return pl.pallas_call(
        body,
        grid=(pl.cdiv(V, BN),),
        in_specs=[pl.BlockSpec((C, BN), lambda i: (0, i))],
        out_specs=pl.BlockSpec((BN // 2, 2 * C), lambda i: (i, 0)),
        out_shape=jax.ShapeDtypeStruct((V // 2, 2 * C), jnp.float32),
    )


@functools.lru_cache(maxsize=None)
def _build_transpose(M, K):
    L = 128
    BM = 512
    nk = K // L
    assert M % BM == 0 and K % L == 0

    def body(in_ref, out_ref):
        a = in_ref[...].reshape(BM, nk, L)
        for ks in range(nk):
            out_ref[pl.ds(ks * L, L), :] = jnp.transpose(a[:, ks, :])

    return pl.pallas_call(
        body,
        grid=(M // BM,),
        in_specs=[pl.BlockSpec((BM * nk, L), lambda i: (i, 0))],
        out_specs=pl.BlockSpec((K, BM), lambda i: (0, i)),
        out_shape=jax.ShapeDtypeStruct((K, M), jnp.float32),
    )


@functools.lru_cache(maxsize=None)
def _build_gather(B, V, D):
    info = plsc.get_sparse_core_info()
    NC, NS = info.num_cores, info.num_subcores
    NW = NC * NS
    assert B % NW == 0
    b_per_w = B // NW
    CH = 256
    NBUF = 4
    assert b_per_w % CH == 0
    n_ch = b_per_w // CH
    assert n_ch % NBUF == 0

    mesh = plsc.VectorSubcoreMesh(core_axis_name="c", subcore_axis_name="s")

    @functools.partial(
        pl.kernel,
        mesh=mesh,
        compiler_params=pltpu.CompilerParams(use_tc_tiling_on_sc=False),
        out_type=jax.ShapeDtypeStruct((B, D), jnp.float32),
        scratch_types=(
            [pltpu.VMEM((b_per_w,), jnp.int32),
             pltpu.VMEM((NBUF, CH, D), jnp.float32)]
            + [pltpu.SemaphoreType.DMA] * (2 * NBUF)
        ),
    )
    def gather_kernel(idx_hbm, table_hbm, out_hbm, idx_v, rows_v, *sems):
        gsems, psems = sems[:NBUF], sems[NBUF:]
        wid = lax.axis_index("s") * NC + lax.axis_index("c")
        base = wid * b_per_w
        pltpu.sync_copy(idx_hbm.at[pl.ds(base, b_per_w)], idx_v)

        def start_gather(j, b):
            pltpu.async_copy(
                table_hbm.at[idx_v.at[pl.ds(j * CH, CH)]], rows_v.at[b], gsems[b])

        def wait_gather(b):
            pltpu.make_async_copy(
                table_hbm.at[pl.ds(0, CH)], rows_v.at[b], gsems[b]).wait()

        def start_put(j, b):
            pltpu.async_copy(
                rows_v.at[b], out_hbm.at[pl.ds(base + j * CH, CH)], psems[b])

        def wait_put(b):
            pltpu.make_async_copy(
                rows_v.at[b], out_hbm.at[pl.ds(0, CH)], psems[b]).wait()

        for j in range(NBUF - 1):
            start_gather(j, j)

        def group(g, carry):
            for b in range(NBUF):
                j = g * NBUF + b
                wait_gather(b)
                start_put(j, b)
                gj = j + NBUF - 1
                gb = (b - 1) % NBUF

                @pl.when(gj < n_ch)
                def _():
                    @pl.when(j > 0)
                    def _():
                        wait_put(gb)
                    start_gather(gj, gb)
            return carry

        lax.fori_loop(0, n_ch // NBUF, group, 0)

        for b in range(NBUF):
            wait_put(b)

    return gather_kernel


def kernel(x, table):
    V, D = table.shape
    N, S = x.shape
    B = N * S
    xf = x.reshape(-1).astype(jnp.int32)
    t_pair = _build_pack_table(D, V)(jnp.transpose(table))
    t_lin = jnp.reshape(t_pair, (V, D))
    res = _build_gather(B, V, D)(xf, t_lin)
    K = S * D
    in2 = jnp.reshape(res, (B * D // 128, 128))
    out2 = _build_transpose(N, K)(in2)
    out3 = jnp.reshape(out2, (S, D, N))
    return jnp.transpose(out3, (2, 0, 1))
